# Initial kernel scaffold; baseline (speedup 1.0000x reference)
import jax
import jax.numpy as jnp
from jax.experimental import pallas as pl


def kernel(x, edge_index, batch, W_lin, b_lin, g0, be0, Wg0, as0, ad0, bg0, g1, be1, Wg1, as1, ad1, bg1, k1, cw1, Wm1, bm1, k2, cw2, Wm2, bm2):
    raise NotImplementedError("write your pallas kernel here")


# speedup vs baseline: 11.9474x; 11.9474x over previous
import functools

import jax
import jax.numpy as jnp
from jax import lax
from jax.experimental import pallas as pl
from jax.experimental.pallas import tpu as pltpu
from jax.experimental.pallas import tpu_sc as plsc

N = 10000
NP = 10240
D = 128
B = 16
E = 320000
ET = E + N
NC = 2
NS = 16
NW = NC * NS
EPW = 10368
EP = NW * EPW
CHK = 4
CHKE = EPW // CHK
KC = 48
NBC = EPW // KC
NBC2 = NBC // 2
RPT = NP // NS

_f32 = jnp.float32


def _bn_proj(h, g, be, Wg, a_s, a_d, rowmask):
    hm = h * rowmask
    m = jnp.sum(hm, axis=0, keepdims=True) * (1.0 / N)
    c = (h - m) * rowmask
    v = jnp.sum(c * c, axis=0, keepdims=True) * (1.0 / N)
    t = (h - m) / jnp.sqrt(v + 1e-5) * g + be
    t = jnp.where(t > 0, t, 0.01 * t)
    xl = lax.dot_general(t, Wg, (((1,), (1,)), ((), ())),
                         preferred_element_type=_f32)
    s_ = jnp.sum(xl * a_s, axis=1, keepdims=True)
    d_ = jnp.sum(xl * a_d, axis=1, keepdims=True)
    return xl, s_, d_


def _ka_body(x_ref, Wl_ref, bl_ref, g_ref, be_ref, Wg_ref, as_ref, ad_ref,
             h_out, xl_out, s_out, d_out):
    rows = lax.broadcasted_iota(jnp.int32, (NP, 1), 0)
    rowmask = (rows < N).astype(_f32)
    x = x_ref[...]
    h = (lax.dot_general(x, Wl_ref[...], (((1,), (1,)), ((), ())),
                         preferred_element_type=_f32) + bl_ref[...]) * rowmask
    h_out[...] = h
    xl, s_, d_ = _bn_proj(h, g_ref[...], be_ref[...], Wg_ref[...],
                          as_ref[...], ad_ref[...], rowmask)
    xl_out[...] = xl
    s_out[...] = s_
    d_out[...] = d_


def _kd_body(h_ref, aggA_ref, aggB_ref, bg_ref, g_ref, be_ref, Wg_ref,
             as_ref, ad_ref, h_out, xl_out, s_out, d_out):
    rows = lax.broadcasted_iota(jnp.int32, (NP, 1), 0)
    rowmask = (rows < N).astype(_f32)
    h = h_ref[...] + (aggA_ref[0] + aggA_ref[1] + aggB_ref[0] + aggB_ref[1]
                      + bg_ref[...]) * rowmask
    h_out[...] = h
    xl, s_, d_ = _bn_proj(h, g_ref[...], be_ref[...], Wg_ref[...],
                          as_ref[...], ad_ref[...], rowmask)
    xl_out[...] = xl
    s_out[...] = s_
    d_out[...] = d_


BLK = 1024
NBK = NP // BLK


def _ke1_body(h_ref, aggA_ref, aggB_ref, bg_ref, batch_ref, kk_ref, kk2_ref,
              cwl_ref, S_out, den1_out, xp_out):
    i = pl.program_id(0)
    rows = i * BLK + lax.broadcasted_iota(jnp.int32, (BLK, 1), 0)
    rowmask = (rows < N).astype(_f32)
    h = h_ref[...] + (aggA_ref[0] + aggA_ref[1] + aggB_ref[0] + aggB_ref[1]
                      + bg_ref[...]) * rowmask

    hh2 = jnp.sum(h * h, axis=1, keepdims=True)
    hkk = lax.dot_general(h, kk_ref[...], (((1,), (1,)), ((), ())),
                          preferred_element_type=_f32)
    d2 = jnp.maximum(hh2 - 2.0 * hkk + kk2_ref[...], 0.0)
    dd = 1.0 / (1.0 + d2)
    j64 = lax.broadcasted_iota(jnp.int32, (64, 16), 0)
    c16 = lax.broadcasted_iota(jnp.int32, (64, 16), 1)
    A = ((j64 < 50) & (j64 // 10 == c16)).astype(_f32)
    Cm = ((j64 < 50) & (j64 % 10 == c16)).astype(_f32)
    headsum = jnp.dot(dd, A, preferred_element_type=_f32)
    inv = 1.0 / jnp.where(headsum == 0, 1.0, headsum)
    invl = lax.dot_general(inv, A, (((1,), (1,)), ((), ())),
                           preferred_element_type=_f32)
    Sh = dd * invl * cwl_ref[...]
    Scomb = jnp.dot(Sh, Cm, preferred_element_type=_f32)
    lane16 = lax.broadcasted_iota(jnp.int32, (BLK, 16), 1)
    lmask = lane16 < 10
    mx = jnp.max(jnp.where(lmask, Scomb, -1e30), axis=1, keepdims=True)
    ex = jnp.where(lmask, jnp.exp(Scomb - mx), 0.0)
    S = ex / jnp.sum(ex, axis=1, keepdims=True)
    S_out[...] = S

    batchv = batch_ref[...]
    onehot = (batchv == lax.broadcasted_iota(jnp.int32, (BLK, 16), 1)).astype(_f32)

    @pl.when(i == 0)
    def _init():
        den1_out[...] = jnp.zeros((16, 16), _f32)
        xp_out[...] = jnp.zeros((256, D), _f32)

    den1_out[...] += lax.dot_general(onehot, S, (((0,), (0,)), ((), ())),
                                     preferred_element_type=_f32)
    j256 = lax.broadcasted_iota(jnp.int32, (16, 256), 1)
    r16 = lax.broadcasted_iota(jnp.int32, (16, 256), 0)
    Aexp = (j256 // 16 == r16).astype(_f32)
    Cexp = (j256 % 16 == r16).astype(_f32)
    oe = jnp.dot(onehot, Aexp, preferred_element_type=_f32)
    se = jnp.dot(S, Cexp, preferred_element_type=_f32)
    xp_out[...] += lax.dot_general(oe * se, h, (((0,), (0,)), ((), ())),
                                   preferred_element_type=_f32)


def _ke2_body(S_ref, batch_ref, den1_ref, xp_ref, Wm1_ref, bm1_ref,
              Wm2_ref, bm2_ref, logp_out, kl_out):
    i = pl.program_id(0)
    rows = i * BLK + lax.broadcasted_iota(jnp.int32, (BLK, 1), 0)
    rowmask = (rows < N).astype(_f32)
    S = S_ref[...]
    batchv = batch_ref[...]
    onehot = (batchv == lax.broadcasted_iota(jnp.int32, (BLK, 16), 1)).astype(_f32)

    den1_pn = jnp.dot(onehot, den1_ref[...], preferred_element_type=_f32)
    P = S * S / jnp.where(den1_pn == 0, 1.0, den1_pn)
    den2 = jnp.sum(P, axis=1, keepdims=True)
    P = P / jnp.where(den2 == 0, 1.0, den2)
    Pc = jnp.maximum(P, 1e-15)
    Sc = jnp.maximum(S, 1e-15)
    kl = jnp.sum(Pc * (jnp.log(Pc) - jnp.log(Sc)) * rowmask) * (1.0 / B)

    @pl.when(i == 0)
    def _init():
        kl_out[...] = jnp.zeros((1, 1), _f32)
    kl_out[...] += kl

    @pl.when(i == NBK - 1)
    def _head():
        x1 = lax.dot_general(xp_ref[...], Wm1_ref[...],
                             (((1,), (1,)), ((), ())),
                             preferred_element_type=_f32) + bm1_ref[...]
        x1 = jnp.where(x1 > 0, x1, 0.01 * x1)
        r256 = lax.broadcasted_iota(jnp.int32, (16, 256), 1)
        b16 = lax.broadcasted_iota(jnp.int32, (16, 256), 0)
        M = ((r256 // 16 == b16) & (r256 % 16 < 10)).astype(_f32)
        x1s = jnp.dot(M, x1, preferred_element_type=_f32)
        x2 = lax.dot_general(x1s, Wm2_ref[...], (((1,), (1,)), ((), ())),
                             preferred_element_type=_f32) + bm2_ref[...]
        l16 = lax.broadcasted_iota(jnp.int32, (16, 16), 1)
        lm = l16 < 10
        mx2 = jnp.max(jnp.where(lm, x2, -1e30), axis=1, keepdims=True)
        lse = jnp.log(jnp.sum(jnp.where(lm, jnp.exp(x2 - mx2), 0.0),
                              axis=1, keepdims=True))
        logp_out[...] = x2 - mx2 - lse


_mesh = plsc.VectorSubcoreMesh(core_axis_name="c", subcore_axis_name="s",
                               num_cores=NC, num_subcores=NS)


def _kb_body(s_hbm, d_hbm, src_hbm, dst_hbm, w_hbm, den_hbm,
             s_v, d_v, src_c, dst_c, w_c, den_l):
    cid = lax.axis_index("c")
    sid = lax.axis_index("s")
    wid = cid * NS + sid
    base = wid * EPW
    pltpu.sync_copy(s_hbm, s_v)
    pltpu.sync_copy(d_hbm, d_v)

    def zero(i, _):
        den_l[pl.ds(i * 16, 16)] = jnp.zeros((16,), _f32)
        return 0
    lax.fori_loop(0, NP // 16, zero, 0)

    lanes = lax.iota(jnp.int32, 16)

    for k in range(CHK):
        cbase = base + k * CHKE
        pltpu.sync_copy(src_hbm.at[pl.ds(cbase, CHKE)], src_c)
        pltpu.sync_copy(dst_hbm.at[pl.ds(cbase, CHKE)], dst_c)

        def step(i, _):
            si = src_c[pl.ds(i * 16, 16)]
            di = dst_c[pl.ds(i * 16, 16)]
            sv = plsc.load_gather(s_v, [si])
            dv = plsc.load_gather(d_v, [di])
            al = sv + dv
            al = jnp.where(al > 0, al, 0.2 * al)
            wv = jnp.exp(al)
            gidx = cbase + i * 16 + lanes
            wv = jnp.where(gidx < ET, wv, 0.0)
            w_c[pl.ds(i * 16, 16)] = wv
            plsc.addupdate_scatter(den_l, [di], wv)
            return 0
        lax.fori_loop(0, CHKE // 16, step, 0)
        pltpu.sync_copy(w_c, w_hbm.at[pl.ds(cbase, CHKE)])

    pltpu.sync_copy(den_l, den_hbm.at[wid])


HNP = NP // 2
ACCR = HNP + 16
RPC = HNP // NS


def _kc_body(src_hbm, dst_hbm, w_hbm, den_hbm, xl_hbm, den_out, agg_hbm,
             src2d, dst2d, coef2d, part1, den_v, rows_v, dstp1, acc_sh):
    cid = lax.axis_index("c")
    sid = lax.axis_index("s")
    wid = cid * NS + sid
    nbase = sid * RPT

    pltpu.sync_copy(src_hbm.at[wid], src2d)
    pltpu.sync_copy(dst_hbm.at[wid], dst2d)
    pltpu.sync_copy(w_hbm.at[wid], coef2d)

    def zden(j, _):
        den_v[pl.ds(nbase + j * 16, 16)] = jnp.zeros((16,), _f32)
        return 0
    lax.fori_loop(0, RPT // 16, zden, 0)
    for w in range(NW):
        pltpu.sync_copy(den_hbm.at[w].at[pl.ds(nbase, RPT)], part1)
        def accw(j, _):
            den_v[pl.ds(nbase + j * 16, 16)] = (
                den_v[pl.ds(nbase + j * 16, 16)] + part1[pl.ds(j * 16, 16)])
            return 0
        lax.fori_loop(0, RPT // 16, accw, 0)
    pltpu.sync_copy(den_v.at[pl.ds(nbase, RPT)],
                    den_out.at[cid].at[pl.ds(nbase, RPT)])

    def zrow(r, _):
        for j in range(D // 16):
            rows_v[r, pl.ds(j * 16, 16)] = jnp.zeros((16,), _f32)
        return 0
    lax.fori_loop(0, KC, zrow, 0)

    def zero_own_slice():
        abase = sid * RPC
        for k in range(RPC // KC):
            pltpu.sync_copy(rows_v, acc_sh.at[pl.ds(abase + k * KC, KC)])
        pltpu.sync_copy(rows_v.at[pl.ds(0, RPC - (RPC // KC) * KC)],
                        acc_sh.at[pl.ds(abase + (RPC // KC) * KC,
                                        RPC - (RPC // KC) * KC)])
        pltpu.sync_copy(rows_v.at[pl.ds(0, 1)], acc_sh.at[pl.ds(HNP + sid, 1)])
    zero_own_slice()

    plsc.subcore_barrier()
    pltpu.sync_copy(den_out.at[cid], den_v)

    def mkcoef(i, _):
        for j in range(KC // 16):
            di = dst2d[i, pl.ds(j * 16, 16)]
            wv = coef2d[i, pl.ds(j * 16, 16)]
            dn = plsc.load_gather(den_v, [di])
            coef2d[i, pl.ds(j * 16, 16)] = wv / (dn + 1e-16)
        return 0
    lax.fori_loop(0, NBC2, mkcoef, 0)

    lanes16 = lax.iota(jnp.int32, 16)
    for p in range(2):
        def blk(i, _):
            pltpu.sync_copy(xl_hbm.at[src2d.at[i]], rows_v)
            for rb in range(KC // 16):
                cv = coef2d[i, pl.ds(rb * 16, 16)]
                di = dst2d[i, pl.ds(rb * 16, 16)] - p * HNP
                ok = (di >= 0) & (di < HNP)
                dstp1[pl.ds(rb * 16, 16)] = jnp.where(ok, di, HNP + lanes16)
                for k in range(16):
                    r = rb * 16 + k
                    c = cv[k]
                    for j in range(D // 16):
                        rows_v[r, pl.ds(j * 16, 16)] = (
                            rows_v[r, pl.ds(j * 16, 16)] * c)
            pltpu.sync_copy(rows_v, acc_sh.at[dstp1], add=True)
            return 0
        lax.fori_loop(0, NBC2, blk, 0)

        plsc.subcore_barrier()
        abase = sid * RPC
        pltpu.sync_copy(acc_sh.at[pl.ds(abase, RPC)],
                        agg_hbm.at[cid].at[pl.ds(p * HNP + abase, RPC)])
        if p == 0:
            def zrow2(r, _):
                for j in range(D // 16):
                    rows_v[r, pl.ds(j * 16, 16)] = jnp.zeros((16,), _f32)
                return 0
            lax.fori_loop(0, KC, zrow2, 0)
            zero_own_slice()
            plsc.subcore_barrier()


def _build(interpret):
    dense4 = [jax.ShapeDtypeStruct((NP, D), _f32),
              jax.ShapeDtypeStruct((NP, D), _f32),
              jax.ShapeDtypeStruct((NP, 1), _f32),
              jax.ShapeDtypeStruct((NP, 1), _f32)]
    ka = pl.pallas_call(_ka_body, out_shape=dense4, interpret=interpret)
    kd = pl.pallas_call(_kd_body, out_shape=dense4, interpret=interpret)
    ke1 = pl.pallas_call(
        _ke1_body,
        grid=(NBK,),
        in_specs=[
            pl.BlockSpec((BLK, D), lambda i: (i, 0)),
            pl.BlockSpec((NC, BLK, D), lambda i: (0, i, 0)),
            pl.BlockSpec((NC, BLK, D), lambda i: (0, i, 0)),
            pl.BlockSpec((1, D), lambda i: (0, 0)),
            pl.BlockSpec((BLK, 1), lambda i: (i, 0)),
            pl.BlockSpec((64, D), lambda i: (0, 0)),
            pl.BlockSpec((1, 64), lambda i: (0, 0)),
            pl.BlockSpec((1, 64), lambda i: (0, 0)),
        ],
        out_specs=[
            pl.BlockSpec((BLK, 16), lambda i: (i, 0)),
            pl.BlockSpec((16, 16), lambda i: (0, 0)),
            pl.BlockSpec((256, D), lambda i: (0, 0)),
        ],
        out_shape=[jax.ShapeDtypeStruct((NP, 16), _f32),
                   jax.ShapeDtypeStruct((16, 16), _f32),
                   jax.ShapeDtypeStruct((256, D), _f32)],
        interpret=interpret)
    ke2 = pl.pallas_call(
        _ke2_body,
        grid=(NBK,),
        in_specs=[
            pl.BlockSpec((BLK, 16), lambda i: (i, 0)),
            pl.BlockSpec((BLK, 1), lambda i: (i, 0)),
            pl.BlockSpec((16, 16), lambda i: (0, 0)),
            pl.BlockSpec((256, D), lambda i: (0, 0)),
            pl.BlockSpec((D, D), lambda i: (0, 0)),
            pl.BlockSpec((1, D), lambda i: (0, 0)),
            pl.BlockSpec((16, D), lambda i: (0, 0)),
            pl.BlockSpec((1, 16), lambda i: (0, 0)),
        ],
        out_specs=[pl.BlockSpec((16, 16), lambda i: (0, 0)),
                   pl.BlockSpec((1, 1), lambda i: (0, 0))],
        out_shape=[jax.ShapeDtypeStruct((16, 16), _f32),
                   jax.ShapeDtypeStruct((1, 1), _f32)],
        interpret=interpret)
    kb = pl.kernel(
        _kb_body,
        out_type=[jax.ShapeDtypeStruct((EP,), _f32),
                  jax.ShapeDtypeStruct((NW, NP), _f32)],
        mesh=_mesh,
        interpret=interpret,
        compiler_params=pltpu.CompilerParams(needs_layout_passes=False),
        scratch_types=[
            pltpu.VMEM((NP,), _f32),
            pltpu.VMEM((NP,), _f32),
            pltpu.VMEM((CHKE,), jnp.int32),
            pltpu.VMEM((CHKE,), jnp.int32),
            pltpu.VMEM((CHKE,), _f32),
            pltpu.VMEM((NP,), _f32),
        ])
    kc = pl.kernel(
        _kc_body,
        out_type=[jax.ShapeDtypeStruct((NC, NP), _f32),
                  jax.ShapeDtypeStruct((NC, NP, D), _f32)],
        mesh=_mesh,
        interpret=interpret,
        compiler_params=pltpu.CompilerParams(needs_layout_passes=False),
        scratch_types=[
            pltpu.VMEM((NBC2, KC), jnp.int32),
            pltpu.VMEM((NBC2, KC), jnp.int32),
            pltpu.VMEM((NBC2, KC), _f32),
            pltpu.VMEM((RPT,), _f32),
            pltpu.VMEM((NP,), _f32),
            pltpu.VMEM((KC, D), _f32),
            pltpu.VMEM((KC,), jnp.int32),
            pltpu.VMEM_SHARED((ACCR, D), _f32),
        ])
    return ka, kd, ke1, ke2, kb, kc


_ka, _kd, _ke1, _ke2, _kb, _kc = _build(False)


def kernel(x, edge_index, batch, W_lin, b_lin, g0, be0, Wg0, as0, ad0, bg0,
           g1, be1, Wg1, as1, ad1, bg1, k1, cw1, Wm1, bm1, k2, cw2, Wm2, bm2):
    row = lambda v: v.reshape(1, -1).astype(_f32)
    xp_ = jnp.pad(x, ((0, NP - N), (0, 0)))
    batchp = jnp.pad(batch.astype(jnp.int32), (0, NP - N),
                     constant_values=B).reshape(NP, 1)
    loops = jnp.arange(N, dtype=jnp.int32)
    zpad = jnp.zeros((EP - ET,), jnp.int32)
    srcp = jnp.concatenate([edge_index[0].astype(jnp.int32), loops, zpad])
    dstp = jnp.concatenate([edge_index[1].astype(jnp.int32), loops, zpad])
    src3 = srcp.reshape(NW, NBC, KC)
    dst3 = dstp.reshape(NW, NBC, KC)
    srcA, srcB = src3[:, :NBC2], src3[:, NBC2:]
    dstA, dstB = dst3[:, :NBC2], dst3[:, NBC2:]

    h, xl, s_, d_ = _ka(xp_, W_lin, row(b_lin), row(g0), row(be0), Wg0,
                        row(as0), row(ad0))
    w_e, denp = _kb(s_.reshape(NP), d_.reshape(NP), srcp, dstp)
    w3 = w_e.reshape(NW, NBC, KC)
    _, aggA = _kc(srcA, dstA, w3[:, :NBC2], denp, xl)
    _, aggB = _kc(srcB, dstB, w3[:, NBC2:], denp, xl)

    h, xl, s_, d_ = _kd(h, aggA, aggB, row(bg0), row(g1), row(be1), Wg1,
                        row(as1), row(ad1))
    w_e, denp = _kb(s_.reshape(NP), d_.reshape(NP), srcp, dstp)
    w3 = w_e.reshape(NW, NBC, KC)
    _, aggA = _kc(srcA, dstA, w3[:, :NBC2], denp, xl)
    _, aggB = _kc(srcB, dstB, w3[:, NBC2:], denp, xl)

    kk = jnp.pad(k1.reshape(50, D), ((0, 14), (0, 0))).astype(_f32)
    kk2 = jnp.sum(kk * kk, axis=1).reshape(1, 64)
    cwl = jnp.pad(jnp.repeat(cw1.astype(_f32), 10), (0, 14)).reshape(1, 64)
    Wm1p = jnp.pad(Wm1.astype(_f32), ((0, 48), (0, 0)))
    bm1p = jnp.pad(bm1.astype(_f32), (0, 48)).reshape(1, 128)
    Wm2p = jnp.pad(Wm2.astype(_f32), ((0, 6), (0, 48)))
    bm2p = jnp.pad(bm2.astype(_f32), (0, 6)).reshape(1, 16)

    S, den1, xp = _ke1(h, aggA, aggB, row(bg1), batchp, kk, kk2, cwl)
    logp16, kl = _ke2(S, batchp, den1, xp, Wm1p, bm1p, Wm2p, bm2p)
    return logp16[:, :10], kl.reshape(())
